# Initial kernel scaffold; baseline (speedup 1.0000x reference)
import jax
import jax.numpy as jnp
from jax.experimental import pallas as pl


def kernel(user, item_p, item_n, mask, users_emb, items_emb, blen_pop):
    raise NotImplementedError("write your pallas kernel here")


# speedup vs baseline: 2.2582x; 2.2582x over previous
import dataclasses
import functools

import jax
import jax.numpy as jnp
from jax import lax
from jax.experimental import pallas as pl
from jax.experimental.pallas import tpu as pltpu
from jax.experimental.pallas import tpu_sc as plsc

NUM_ITEMS = 1000
EMB = 32
N = 4096 * 50
NC, NS, L = 2, 16, 16
NW = NC * NS
PER_W = N // NW
WIN = 128
NWIN = PER_W // WIN
GRP = WIN // L
EPS = 1e-10

_mesh = plsc.VectorSubcoreMesh(core_axis_name="c", subcore_axis_name="s")

_cp = pltpu.CompilerParams(use_tc_tiling_on_sc=False)
if "needs_layout_passes" in pltpu.CompilerParams.__dataclass_fields__:
    _cp = dataclasses.replace(_cp, needs_layout_passes=False)


@functools.partial(
    pl.kernel,
    compiler_params=_cp,
    out_type=jax.ShapeDtypeStruct((N,), jnp.float32),
    mesh=_mesh,
    scratch_types=[
        pltpu.VMEM((NUM_ITEMS * EMB,), jnp.float32),
        pltpu.VMEM((PER_W,), jnp.int32),
        pltpu.VMEM((PER_W,), jnp.int32),
        pltpu.VMEM((PER_W,), jnp.int32),
        pltpu.VMEM((WIN, EMB), jnp.float32),
        pltpu.VMEM((WIN,), jnp.float32),
        pltpu.SemaphoreType.DMA,
    ],
)
def _sc_scores(user_hbm, ip_hbm, in_hbm, uemb_hbm, iemb_hbm, out_hbm,
               items_v, uidx_v, pidx_v, nidx_v, urows_v, s_v, sem):
    wid = lax.axis_index("s") * NC + lax.axis_index("c")
    base0 = wid * PER_W
    pltpu.sync_copy(iemb_hbm, items_v)
    pltpu.sync_copy(user_hbm.at[pl.ds(base0, PER_W)], uidx_v)
    pltpu.sync_copy(ip_hbm.at[pl.ds(base0, PER_W)], pidx_v)
    pltpu.sync_copy(in_hbm.at[pl.ds(base0, PER_W)], nidx_v)
    lanes = lax.iota(jnp.int32, L)

    @pl.loop(0, NWIN)
    def _window(w):
        off = w * WIN
        pltpu.async_copy(uemb_hbm.at[uidx_v.at[pl.ds(off, WIN)]],
                         urows_v, sem).wait()

        @pl.loop(0, GRP)
        def _group(g):
            pv = pidx_v[pl.ds(off + g * L, L)] * EMB
            nv = nidx_v[pl.ds(off + g * L, L)] * EMB
            rowv = g * L + lanes
            acc = jnp.zeros((L,), jnp.float32)
            colv = jnp.zeros((L,), jnp.int32)
            for k in range(EMB):
                u = plsc.load_gather(urows_v, [rowv, colv])
                p = plsc.load_gather(items_v, [pv])
                n = plsc.load_gather(items_v, [nv])
                acc = acc + u * (p - n)
                if k + 1 < EMB:
                    pv = pv + 1
                    nv = nv + 1
                    colv = colv + 1
            s_v[pl.ds(g * L, L)] = acc

        pltpu.sync_copy(s_v, out_hbm.at[pl.ds(base0 + off, WIN)])


def _tc_loss(scores):
    def body(s_ref, o_ref):
        x = s_ref[...]
        sig = 1.0 / (1.0 + jnp.exp(-x))
        o_ref[0, 0] = -jnp.sum(jnp.log(sig + EPS)) * (1.0 / N)

    out = pl.pallas_call(
        body,
        out_shape=jax.ShapeDtypeStruct((1, 1), jnp.float32),
        out_specs=pl.BlockSpec(memory_space=pltpu.SMEM),
    )(scores)
    return out[0, 0]


def kernel(user, item_p, item_n, mask, users_emb, items_emb, blen_pop):
    scores = _sc_scores(user.reshape(N), item_p.reshape(N),
                        item_n.reshape(N), users_emb,
                        items_emb.reshape(NUM_ITEMS * EMB))
    return _tc_loss(scores.reshape(N // 128, 128))


# speedup vs baseline: 3.2014x; 1.4176x over previous
import dataclasses
import functools

import jax
import jax.numpy as jnp
from jax import lax
from jax.experimental import pallas as pl
from jax.experimental.pallas import tpu as pltpu
from jax.experimental.pallas import tpu_sc as plsc

NUM_ITEMS = 1000
ITEM_PAD = 1009
EMB = 32
N = 4096 * 50
NC, NS, L = 2, 16, 16
NW = NC * NS
PER_W = N // NW
WIN = 128
WIN_PAD = 129
NWIN = PER_W // WIN
GRP = WIN // L
EPS = 1e-10

_mesh = plsc.VectorSubcoreMesh(core_axis_name="c", subcore_axis_name="s")

_cp = pltpu.CompilerParams(use_tc_tiling_on_sc=False)
if "needs_layout_passes" in pltpu.CompilerParams.__dataclass_fields__:
    _cp = dataclasses.replace(_cp, needs_layout_passes=False)


@functools.partial(
    pl.kernel,
    compiler_params=_cp,
    out_type=jax.ShapeDtypeStruct((N,), jnp.float32),
    mesh=_mesh,
    scratch_types=[
        pltpu.VMEM((EMB, ITEM_PAD), jnp.float32),
        pltpu.VMEM((PER_W,), jnp.int32),
        pltpu.VMEM((PER_W,), jnp.int32),
        pltpu.VMEM((PER_W,), jnp.int32),
        pltpu.VMEM((WIN, EMB), jnp.float32),
        pltpu.VMEM((WIN, EMB), jnp.float32),
        pltpu.VMEM((EMB, WIN_PAD), jnp.float32),
        pltpu.VMEM((PER_W,), jnp.float32),
        pltpu.SemaphoreType.DMA,
        pltpu.SemaphoreType.DMA,
    ],
)
def _sc_scores(user_hbm, ip_hbm, in_hbm, uemb_hbm, itemsT_hbm, out_hbm,
               items_v, uidx_v, pidx_v, nidx_v, ubufA, ubufB, ut_v, s_v,
               semA, semB):
    wid = lax.axis_index("s") * NC + lax.axis_index("c")
    base0 = wid * PER_W
    pltpu.sync_copy(itemsT_hbm, items_v)
    pltpu.sync_copy(user_hbm.at[pl.ds(base0, PER_W)], uidx_v)
    pltpu.sync_copy(ip_hbm.at[pl.ds(base0, PER_W)], pidx_v)
    pltpu.sync_copy(in_hbm.at[pl.ds(base0, PER_W)], nidx_v)
    iota = lax.iota(jnp.int32, L)
    iota16 = iota + L

    def gather(w, ubuf, sem):
        return pltpu.make_async_copy(
            uemb_hbm.at[uidx_v.at[pl.ds(w * WIN, WIN)]], ubuf, sem)

    def process(w, ubuf, sem):
        gather(w, ubuf, sem).wait()

        @pl.loop(0, WIN // 8)
        def _t(t):
            for j in range(8):
                i = t * 8 + j
                ci = jnp.zeros((L,), jnp.int32) + i
                plsc.store_scatter(ut_v, [iota, ci], ubuf[i, pl.ds(0, L)])
                plsc.store_scatter(ut_v, [iota16, ci], ubuf[i, pl.ds(L, L)])

        @pl.when(w + 2 < NWIN)
        def _():
            gather(w + 2, ubuf, sem).start()

        @pl.loop(0, GRP)
        def _group(g):
            off = w * WIN + g * L
            pv = pidx_v[pl.ds(off, L)]
            nv = nidx_v[pl.ds(off, L)]
            acc = jnp.zeros((L,), jnp.float32)
            for k in range(EMB):
                u = ut_v[k, pl.ds(g * L, L)]
                p = plsc.load_gather(items_v.at[k], [pv])
                n = plsc.load_gather(items_v.at[k], [nv])
                acc = acc + u * (p - n)
            s_v[pl.ds(off, L)] = acc

    gather(0, ubufA, semA).start()
    gather(1, ubufB, semB).start()

    @pl.loop(0, NWIN, step=2)
    def _window(w):
        process(w, ubufA, semA)
        process(w + 1, ubufB, semB)

    pltpu.sync_copy(s_v, out_hbm.at[pl.ds(base0, PER_W)])


def _tc_loss(scores):
    def body(s_ref, o_ref):
        x = s_ref[...]
        sig = 1.0 / (1.0 + jnp.exp(-x))
        o_ref[0, 0] = -jnp.sum(jnp.log(sig + EPS)) * (1.0 / N)

    out = pl.pallas_call(
        body,
        out_shape=jax.ShapeDtypeStruct((1, 1), jnp.float32),
        out_specs=pl.BlockSpec(memory_space=pltpu.SMEM),
    )(scores)
    return out[0, 0]


def kernel(user, item_p, item_n, mask, users_emb, items_emb, blen_pop):
    items_T = jnp.pad(items_emb.T, ((0, 0), (0, ITEM_PAD - NUM_ITEMS)))
    scores = _sc_scores(user.reshape(N), item_p.reshape(N),
                        item_n.reshape(N), users_emb, items_T)
    return _tc_loss(scores.reshape(N // 128, 128))
